# baseline (speedup 1.0000x reference)
import dataclasses
import functools

import jax
import jax.numpy as jnp
from jax import lax
from jax.experimental import pallas as pl
from jax.experimental.pallas import tpu as pltpu
from jax.experimental.pallas import tpu_sc as plsc

_S = 8
_WIN = 64
_NW = 32


def _geom_body(L, Tmax, ds_ref, mult_ref, packed_ref, base_ref):
    f32 = jnp.float32
    ds2 = ds_ref[...]
    mult = mult_ref[0, 0]
    dsf = ds2.astype(f32)
    d = jnp.maximum(jnp.floor(dsf * mult), 1.0)
    step = jnp.where(ds2 > 0, d, 0.0)

    u_io = lax.broadcasted_iota(jnp.int32, (Tmax, Tmax), 0)
    t_io = lax.broadcasted_iota(jnp.int32, (Tmax, Tmax), 1)
    step_bf = step.astype(jnp.bfloat16)
    upper = (u_io <= t_io).astype(jnp.bfloat16)
    ends = lax.dot_general(step_bf, upper, (((1,), (0,)), ((), ())),
                           preferred_element_type=f32)
    starts = ends - step

    tfloor = (t_io // _S) * _S
    before = (u_io < tfloor).astype(jnp.bfloat16)
    wstart = lax.dot_general(step_bf, before, (((1,), (0,)), ((), ())),
                             preferred_element_type=f32)
    wbase = (wstart.astype(jnp.int32) // 8) * 8

    b_io = lax.broadcasted_iota(jnp.int32, ds2.shape, 0)
    lofs = starts.astype(jnp.int32) - wbase
    width = step.astype(jnp.int32)
    packed_ref[...] = lofs | (width << 8)
    base_ref[...] = b_io * L + wbase


def _geometry(ds, mult, L):
    B, Tmax = ds.shape
    spec = pl.BlockSpec((B, Tmax), lambda: (0, 0))
    return pl.pallas_call(
        functools.partial(_geom_body, L, Tmax),
        in_specs=[spec, pl.BlockSpec((1, 1), lambda: (0, 0))],
        out_specs=[spec, spec],
        out_shape=[
            jax.ShapeDtypeStruct((B, Tmax), jnp.int32),
            jax.ShapeDtypeStruct((B, Tmax), jnp.int32),
        ],
    )(ds, mult)


def _sc_avg(meta, hs2, nseg):
    nchunk = meta.shape[0]
    d = hs2.shape[1]
    cpw = nchunk // _NW
    ngrp = d // 16
    nb = 3

    mesh = plsc.VectorSubcoreMesh(core_axis_name="c", subcore_axis_name="s")

    @functools.partial(
        pl.kernel,
        mesh=mesh,
        out_type=jax.ShapeDtypeStruct((nseg, d), jnp.float32),
        scratch_types=(
            [pltpu.VMEM((meta.shape[1],), jnp.int32)] * nb
            + [pltpu.VMEM((_WIN, d), jnp.float32)] * nb
            + [pltpu.VMEM((_S, d), jnp.float32)] * nb
            + [pltpu.SemaphoreType.DMA] * (3 * nb)
        ),
    )
    def run(meta_hbm, hs_hbm, avg_hbm, m0, m1, m2, slab0, slab1, slab2,
            out0, out1, out2, sf0, sf1, sf2, sg0, sg1, sg2, so0, so1, so2):
        m_b = [m0, m1, m2]
        slab_b = [slab0, slab1, slab2]
        out_b = [out0, out1, out2]
        sf = [sf0, sf1, sf2]
        sg = [sg0, sg1, sg2]
        so = [so0, so1, so2]
        wid = lax.axis_index("s") * 2 + lax.axis_index("c")
        base = wid * cpw

        def fetch(c2, b):
            pltpu.async_copy(meta_hbm.at[c2], m_b[b], sf[b])

        def wait_fetch(c2, b):
            pltpu.make_async_copy(meta_hbm.at[c2], m_b[b], sf[b]).wait()

        def base_of(b):
            return pl.multiple_of(m_b[b][pl.ds(0, 16)][8], 8)

        def issue_window(b):
            pltpu.async_copy(hs_hbm.at[pl.ds(base_of(b), _WIN)], slab_b[b],
                             sg[b])

        def wait_window(b):
            pltpu.make_async_copy(hs_hbm.at[pl.ds(base_of(b), _WIN)],
                                  slab_b[b], sg[b]).wait()

        def wait_out(c_prev, b):
            pltpu.make_async_copy(out_b[b],
                                  avg_hbm.at[pl.ds(c_prev * _S, _S)],
                                  so[b]).wait()

        def compute_store(c, b, prev):
            if prev is True:
                wait_out(c - nb, b)
            elif prev is not None:
                pl.when(prev)(lambda: wait_out(c - nb, b))
            slab = slab_b[b]
            out_v = out_b[b]
            mvec = m_b[b][pl.ds(0, 16)]

            for jj in range(_S):
                packed = mvec[jj]
                s_j = packed & 0xFF
                w_j = packed >> 8
                wf = jnp.full((16,), jnp.maximum(w_j, 1),
                              jnp.int32).astype(jnp.float32)
                recip = 1.0 / wf

                zero = jnp.zeros((16,), jnp.float32)

                def body(k, accs):
                    row = slab.at[s_j + k]
                    return tuple(accs[g] + row[pl.ds(g * 16, 16)]
                                 for g in range(ngrp))

                accs = lax.fori_loop(0, w_j, body, (zero,) * ngrp)
                for g in range(ngrp):
                    out_v.at[jj][pl.ds(g * 16, 16)] = accs[g] * recip

            pltpu.async_copy(out_v, avg_hbm.at[pl.ds(c * _S, _S)], so[b])

        fetch(base, 0)
        fetch(base + 1, 1)
        wait_fetch(base, 0)
        issue_window(0)

        @pl.loop(0, (cpw - 2) // nb)
        def _grp(j):
            for i in range(nb):
                c = base + nb * j + i
                bi, bn, bf = i, (i + 1) % nb, (i + 2) % nb
                wait_fetch(c + 1, bn)
                issue_window(bn)
                fetch(c + 2, bf)
                wait_window(bi)
                compute_store(c, bi, j > 0)

        c = base + cpw - 2
        wait_fetch(c + 1, (cpw - 1) % nb)
        issue_window((cpw - 1) % nb)
        wait_window((cpw - 2) % nb)
        compute_store(c, (cpw - 2) % nb, True)
        wait_window((cpw - 1) % nb)
        compute_store(c + 1, (cpw - 1) % nb, True)

        for r in range(cpw - nb, cpw):
            wait_out(base + r, r % nb)

    return run(meta, hs2)


def _mlp_body(avg_ref, w1_ref, b1_ref, w2_ref, b2_ref, out_ref):
    f32 = jnp.float32
    h = lax.dot_general(avg_ref[...], w1_ref[...], (((1,), (1,)), ((), ())),
                        preferred_element_type=f32)
    h = jnp.maximum(h + b1_ref[...][0][None, :], 0.0)
    o = lax.dot_general(h, w2_ref[...], (((1,), (1,)), ((), ())),
                        preferred_element_type=f32)
    out_ref[...] = jnp.maximum(o + b2_ref[...][0][None, :], 0.0)


def _mlp(avg, W1, b1, W2, b2):
    n, d = avg.shape
    h = W2.shape[0]
    b1r = b1.reshape(1, -1)
    b2r = b2.reshape(1, -1)
    blk = 1024
    return pl.pallas_call(
        _mlp_body,
        grid=(n // blk,),
        in_specs=[
            pl.BlockSpec((blk, d), lambda i: (i, 0)),
            pl.BlockSpec(W1.shape, lambda i: (0, 0)),
            pl.BlockSpec(b1r.shape, lambda i: (0, 0)),
            pl.BlockSpec(W2.shape, lambda i: (0, 0)),
            pl.BlockSpec(b2r.shape, lambda i: (0, 0)),
        ],
        out_specs=pl.BlockSpec((blk, h), lambda i: (i, 0)),
        out_shape=jax.ShapeDtypeStruct((n, h), jnp.float32),
        compiler_params=pltpu.CompilerParams(
            dimension_semantics=("arbitrary",),
        ),
    )(avg, W1, b1r, W2, b2r)


def kernel(hs, ds, Lmax, W1, b1, W2, b2):
    B, L, D = hs.shape
    Tmax = ds.shape[1]
    H = W2.shape[0]
    mult = (jnp.float32(L) / jnp.asarray(Lmax, jnp.float32)).reshape(1, 1)

    packed, baseg = _geometry(ds, mult, L)

    nchunk = B * Tmax // _S
    meta = jnp.concatenate(
        [packed.reshape(nchunk, _S), baseg.reshape(nchunk, _S)],
        axis=1)

    avg = _sc_avg(meta, hs.reshape(B * L, D), B * Tmax)
    out = _mlp(avg, W1, b1, W2, b2)
    return out.reshape(B, Tmax, H)


# speedup vs baseline: 1.4340x; 1.4340x over previous
import dataclasses
import functools

import jax
import jax.numpy as jnp
from jax import lax
from jax.experimental import pallas as pl
from jax.experimental.pallas import tpu as pltpu
from jax.experimental.pallas import tpu_sc as plsc

_S = 8
_WIN = 64
_NW = 32


def _geom_body(L, Tmax, ds_ref, mult_ref, packed_ref, base_ref):
    f32 = jnp.float32
    ds2 = ds_ref[...]
    mult = mult_ref[0, 0]
    dsf = ds2.astype(f32)
    d = jnp.maximum(jnp.floor(dsf * mult), 1.0)
    step = jnp.where(ds2 > 0, d, 0.0)

    u_io = lax.broadcasted_iota(jnp.int32, (Tmax, Tmax), 0)
    t_io = lax.broadcasted_iota(jnp.int32, (Tmax, Tmax), 1)
    step_bf = step.astype(jnp.bfloat16)
    upper = (u_io <= t_io).astype(jnp.bfloat16)
    ends = lax.dot_general(step_bf, upper, (((1,), (0,)), ((), ())),
                           preferred_element_type=f32)
    starts = ends - step

    tfloor = (t_io // _S) * _S
    before = (u_io < tfloor).astype(jnp.bfloat16)
    wstart = lax.dot_general(step_bf, before, (((1,), (0,)), ((), ())),
                             preferred_element_type=f32)
    wbase = (wstart.astype(jnp.int32) // 8) * 8

    b_io = lax.broadcasted_iota(jnp.int32, ds2.shape, 0)
    lofs = starts.astype(jnp.int32) - wbase
    width = step.astype(jnp.int32)
    packed_ref[...] = lofs | (width << 8)
    base_ref[...] = b_io * L + wbase


def _geometry(ds, mult, L):
    B, Tmax = ds.shape
    spec = pl.BlockSpec((B, Tmax), lambda: (0, 0))
    return pl.pallas_call(
        functools.partial(_geom_body, L, Tmax),
        in_specs=[spec, pl.BlockSpec((1, 1), lambda: (0, 0))],
        out_specs=[spec, spec],
        out_shape=[
            jax.ShapeDtypeStruct((B, Tmax), jnp.int32),
            jax.ShapeDtypeStruct((B, Tmax), jnp.int32),
        ],
    )(ds, mult)


def _sc_avg(meta, hs2, nseg):
    nchunk = meta.shape[0]
    d = hs2.shape[1]
    cpw = nchunk // _NW
    ngrp = d // 16
    nb = 3

    mesh = plsc.VectorSubcoreMesh(core_axis_name="c", subcore_axis_name="s")

    @functools.partial(
        pl.kernel,
        mesh=mesh,
        out_type=jax.ShapeDtypeStruct((nseg, d), jnp.float32),
        scratch_types=(
            [pltpu.VMEM((meta.shape[1],), jnp.int32)] * nb
            + [pltpu.VMEM((_WIN, d), jnp.float32)] * nb
            + [pltpu.VMEM((_S, d), jnp.float32)] * nb
            + [pltpu.SemaphoreType.DMA] * (3 * nb)
        ),
    )
    def run(meta_hbm, hs_hbm, avg_hbm, m0, m1, m2, slab0, slab1, slab2,
            out0, out1, out2, sf0, sf1, sf2, sg0, sg1, sg2, so0, so1, so2):
        m_b = [m0, m1, m2]
        slab_b = [slab0, slab1, slab2]
        out_b = [out0, out1, out2]
        sf = [sf0, sf1, sf2]
        sg = [sg0, sg1, sg2]
        so = [so0, so1, so2]
        wid = lax.axis_index("s") * 2 + lax.axis_index("c")
        base = wid * cpw

        def fetch(c2, b):
            pltpu.async_copy(meta_hbm.at[c2], m_b[b], sf[b])

        def wait_fetch(c2, b):
            pltpu.make_async_copy(meta_hbm.at[c2], m_b[b], sf[b]).wait()

        def base_of(b):
            return pl.multiple_of(m_b[b][pl.ds(128, 16)][0], 8)

        def issue_window(b):
            pltpu.async_copy(hs_hbm.at[pl.ds(base_of(b), _WIN)], slab_b[b],
                             sg[b])

        def wait_window(b):
            pltpu.make_async_copy(hs_hbm.at[pl.ds(base_of(b), _WIN)],
                                  slab_b[b], sg[b]).wait()

        def wait_out(c_prev, b):
            pltpu.make_async_copy(out_b[b],
                                  avg_hbm.at[pl.ds(c_prev * _S, _S)],
                                  so[b]).wait()

        def compute_store(c, b, prev):
            if prev is True:
                wait_out(c - nb, b)
            elif prev is not None:
                pl.when(prev)(lambda: wait_out(c - nb, b))
            slab = slab_b[b]
            out_v = out_b[b]
            mv = m_b[b]

            @pl.loop(0, _S)
            def _seg(jj):
                packed = mv[pl.ds(jj * 16, 16)][0]
                s_j = packed & 0xFF
                w_j = packed >> 8
                wf = jnp.full((16,), jnp.maximum(w_j, 1),
                              jnp.int32).astype(jnp.float32)
                recip = 1.0 / wf

                zero = jnp.zeros((16,), jnp.float32)

                def body(k, accs):
                    row = slab.at[s_j + k]
                    return tuple(accs[g] + row[pl.ds(g * 16, 16)]
                                 for g in range(ngrp))

                accs = lax.fori_loop(0, w_j, body, (zero,) * ngrp)
                for g in range(ngrp):
                    out_v.at[jj][pl.ds(g * 16, 16)] = accs[g] * recip

            pltpu.async_copy(out_v, avg_hbm.at[pl.ds(c * _S, _S)], so[b])

        fetch(base, 0)
        fetch(base + 1, 1)
        wait_fetch(base, 0)
        issue_window(0)

        @pl.loop(0, (cpw - 2) // nb)
        def _grp(j):
            for i in range(nb):
                c = base + nb * j + i
                bi, bn, bf = i, (i + 1) % nb, (i + 2) % nb
                wait_fetch(c + 1, bn)
                issue_window(bn)
                fetch(c + 2, bf)
                wait_window(bi)
                compute_store(c, bi, j > 0)

        c = base + cpw - 2
        wait_fetch(c + 1, (cpw - 1) % nb)
        issue_window((cpw - 1) % nb)
        wait_window((cpw - 2) % nb)
        compute_store(c, (cpw - 2) % nb, True)
        wait_window((cpw - 1) % nb)
        compute_store(c + 1, (cpw - 1) % nb, True)

        for r in range(cpw - nb, cpw):
            wait_out(base + r, r % nb)

    return run(meta, hs2)


def _mlp_body(avg_ref, w1_ref, b1_ref, w2_ref, b2_ref, out_ref):
    f32 = jnp.float32
    h = lax.dot_general(avg_ref[...], w1_ref[...], (((1,), (1,)), ((), ())),
                        preferred_element_type=f32)
    h = jnp.maximum(h + b1_ref[...][0][None, :], 0.0)
    o = lax.dot_general(h, w2_ref[...], (((1,), (1,)), ((), ())),
                        preferred_element_type=f32)
    out_ref[...] = jnp.maximum(o + b2_ref[...][0][None, :], 0.0)


def _mlp(avg, W1, b1, W2, b2):
    n, d = avg.shape
    h = W2.shape[0]
    b1r = b1.reshape(1, -1)
    b2r = b2.reshape(1, -1)
    blk = 1024
    return pl.pallas_call(
        _mlp_body,
        grid=(n // blk,),
        in_specs=[
            pl.BlockSpec((blk, d), lambda i: (i, 0)),
            pl.BlockSpec(W1.shape, lambda i: (0, 0)),
            pl.BlockSpec(b1r.shape, lambda i: (0, 0)),
            pl.BlockSpec(W2.shape, lambda i: (0, 0)),
            pl.BlockSpec(b2r.shape, lambda i: (0, 0)),
        ],
        out_specs=pl.BlockSpec((blk, h), lambda i: (i, 0)),
        out_shape=jax.ShapeDtypeStruct((n, h), jnp.float32),
        compiler_params=pltpu.CompilerParams(
            dimension_semantics=("arbitrary",),
        ),
    )(avg, W1, b1r, W2, b2r)


def kernel(hs, ds, Lmax, W1, b1, W2, b2):
    B, L, D = hs.shape
    Tmax = ds.shape[1]
    H = W2.shape[0]
    mult = (jnp.float32(L) / jnp.asarray(Lmax, jnp.float32)).reshape(1, 1)

    packed, baseg = _geometry(ds, mult, L)

    nchunk = B * Tmax // _S

    def expand(x):
        return jnp.broadcast_to(
            x.reshape(nchunk, _S)[:, :, None], (nchunk, _S, 16)
        ).reshape(nchunk, _S * 16)

    meta = jnp.concatenate(
        [expand(packed), expand(baseg)], axis=1)

    avg = _sc_avg(meta, hs.reshape(B * L, D), B * Tmax)
    out = _mlp(avg, W1, b1, W2, b2)
    return out.reshape(B, Tmax, H)
